# baseline (speedup 1.0000x reference)
import jax
import jax.numpy as jnp
from jax import lax
from jax.experimental import pallas as pl
from jax.experimental.pallas import tpu as pltpu
from jax.experimental.pallas import tpu_sc as plsc

_DIM = 16
_NC = 2
_NS = 16
_NW = _NC * _NS
_K = 128
_NBUF = 4


def _gather_body(idx_hbm, table_hbm, out_hbm, idx_v, rows_v, *sems):
    gsem = sems[:_NBUF]
    ssem = sems[_NBUF:]
    nj = idx_v.shape[0]
    wid = lax.axis_index("s") * _NC + lax.axis_index("c")
    base = wid * (nj * _K)

    pltpu.sync_copy(idx_hbm.at[wid], idx_v)

    def fire_gather(b, j):
        pltpu.async_copy(table_hbm.at[idx_v.at[j]], rows_v.at[b], gsem[b])

    def wait_gather(b, j):
        pltpu.make_async_copy(table_hbm.at[idx_v.at[j]], rows_v.at[b], gsem[b]).wait()

    def fire_scatter(b, j):
        pltpu.async_copy(rows_v.at[b], out_hbm.at[pl.ds(base + j * _K, _K)], ssem[b])

    def wait_scatter(b, j):
        pltpu.make_async_copy(rows_v.at[b], out_hbm.at[pl.ds(base + j * _K, _K)], ssem[b]).wait()

    for b in range(_NBUF):
        fire_gather(b, b)

    @pl.loop(0, nj - _NBUF, step=_NBUF)
    def _round(j0):
        for b in range(_NBUF):
            wait_gather(b, j0 + b)
            fire_scatter(b, j0 + b)
        for b in range(_NBUF):
            wait_scatter(b, j0 + b)
            fire_gather(b, j0 + b + _NBUF)

    for b in range(_NBUF):
        wait_gather(b, nj - _NBUF + b)
        fire_scatter(b, nj - _NBUF + b)
    for b in range(_NBUF):
        wait_scatter(b, nj - _NBUF + b)


@jax.jit
def _run(idx3, weight):
    nw, nj, k = idx3.shape
    n = nw * nj * k
    f = pl.kernel(
        _gather_body,
        out_type=jax.ShapeDtypeStruct((n, _DIM), jnp.int32),
        mesh=plsc.VectorSubcoreMesh(core_axis_name="c", subcore_axis_name="s"),
        scratch_types=[
            pltpu.VMEM((nj, k), jnp.int32),
            pltpu.VMEM((_NBUF, k, _DIM), jnp.int32),
        ] + [pltpu.SemaphoreType.DMA] * (2 * _NBUF),
        compiler_params=pltpu.CompilerParams(use_tc_tiling_on_sc=False),
    )
    return f(idx3, weight)


def kernel(input, weight):
    b, h = input.shape
    n = b * h
    assert n % (_NW * _K) == 0
    idx3 = input.astype(jnp.int32).reshape(_NW, n // (_NW * _K), _K)
    nrows, dim = weight.shape
    w32 = jax.lax.bitcast_convert_type(
        weight.reshape(nrows, dim // 2, 2), jnp.int32)
    out32 = _run(idx3, w32)
    out = jax.lax.bitcast_convert_type(out32, jnp.bfloat16)
    return out.reshape(b, h, 2 * _DIM)


# speedup vs baseline: 1.0046x; 1.0046x over previous
import jax
import jax.numpy as jnp
from jax import lax
from jax.experimental import pallas as pl
from jax.experimental.pallas import tpu as pltpu
from jax.experimental.pallas import tpu_sc as plsc

_DIM = 16
_NC = 2
_NS = 16
_NW = _NC * _NS
_K = 128
_NBUF = 8
_D = 4


def _gather_body(idx_hbm, table_hbm, out_hbm, idx_v, rows_v, *sems):
    gsem = sems[:_NBUF]
    ssem = sems[_NBUF:]
    nj = idx_v.shape[0]
    wid = lax.axis_index("s") * _NC + lax.axis_index("c")
    base = wid * (nj * _K)

    pltpu.sync_copy(idx_hbm.at[wid], idx_v)

    def fire_gather(b, j):
        pltpu.async_copy(table_hbm.at[idx_v.at[j]], rows_v.at[b], gsem[b])

    def wait_gather(b, j):
        pltpu.make_async_copy(table_hbm.at[idx_v.at[j]], rows_v.at[b], gsem[b]).wait()

    def fire_scatter(b, j):
        pltpu.async_copy(rows_v.at[b], out_hbm.at[pl.ds(base + j * _K, _K)], ssem[b])

    def wait_scatter(b, j):
        pltpu.make_async_copy(rows_v.at[b], out_hbm.at[pl.ds(base + j * _K, _K)], ssem[b]).wait()

    for j in range(_D):
        fire_gather(j % _NBUF, j)

    def round_(j0, first=False, last=False):
        for b in range(_NBUF):
            j = j0 + b
            wait_gather(b, j)
            fire_scatter(b, j)
            bf = (b + _D) % _NBUF
            if not first or b >= _NBUF - _D:
                wait_scatter(bf, j + _D - _NBUF)
            if not last or b < _NBUF - _D:
                fire_gather(bf, j + _D)

    round_(0, first=True)

    @pl.loop(_NBUF, nj - _NBUF, step=_NBUF)
    def _r(j0):
        round_(j0)

    round_(nj - _NBUF, last=True)
    for b in range(_NBUF - _D, _NBUF):
        wait_scatter(b, nj - _NBUF + b)


@jax.jit
def _run(idx3, weight):
    nw, nj, k = idx3.shape
    n = nw * nj * k
    f = pl.kernel(
        _gather_body,
        out_type=jax.ShapeDtypeStruct((n, _DIM), jnp.int32),
        mesh=plsc.VectorSubcoreMesh(core_axis_name="c", subcore_axis_name="s"),
        scratch_types=[
            pltpu.VMEM((nj, k), jnp.int32),
            pltpu.VMEM((_NBUF, k, _DIM), jnp.int32),
        ] + [pltpu.SemaphoreType.DMA] * (2 * _NBUF),
        compiler_params=pltpu.CompilerParams(use_tc_tiling_on_sc=False),
    )
    return f(idx3, weight)


def kernel(input, weight):
    b, h = input.shape
    n = b * h
    assert n % (_NW * _K) == 0
    idx3 = input.astype(jnp.int32).reshape(_NW, n // (_NW * _K), _K)
    nrows, dim = weight.shape
    w32 = jax.lax.bitcast_convert_type(
        weight.reshape(nrows, dim // 2, 2), jnp.int32)
    out32 = _run(idx3, w32)
    out = jax.lax.bitcast_convert_type(out32, jnp.bfloat16)
    return out.reshape(b, h, 2 * _DIM)


# speedup vs baseline: 1.5171x; 1.5101x over previous
import jax
import jax.numpy as jnp
from jax import lax
from jax.experimental import pallas as pl
from jax.experimental.pallas import tpu as pltpu
from jax.experimental.pallas import tpu_sc as plsc

_DIM = 16
_NC = 2
_NS = 16
_NW = _NC * _NS
_NBUF = 8
_D = 4


def _gather_body(idx_hbm, table_hbm, out_hbm, idx_v, bufg, bufs, *sems):
    gsem = sems[:_NBUF]
    ssem = sems[_NBUF:]
    nrows_w = idx_v.shape[0]
    hist = idx_v.shape[1]
    nj = nrows_w
    wid = lax.axis_index("s") * _NC + lax.axis_index("c")
    wr0 = wid * nrows_w


    pltpu.sync_copy(idx_hbm.at[pl.ds(wr0, nrows_w), :], idx_v)

    def fire_gather(b, j):
        pltpu.async_copy(table_hbm.at[idx_v.at[j]], bufg.at[b], gsem[b])

    def wait_gather(b, j):
        pltpu.make_async_copy(table_hbm.at[idx_v.at[j]], bufg.at[b], gsem[b]).wait()

    def fire_scatter(b, j):
        pltpu.async_copy(bufs.at[b], out_hbm.at[wr0 + j], ssem[b])

    def wait_scatter(b, j):
        pltpu.make_async_copy(bufs.at[b], out_hbm.at[wr0 + j], ssem[b]).wait()

    def relay(b):
        for i in range(hist):
            bufs[b, i, :] = plsc.bitcast(bufg[b, i, :], jnp.bfloat16)

    for j in range(_D):
        fire_gather(j % _NBUF, j)

    def chunk(j, b, first=False, last=False):
        wait_gather(b, j)
        if not first:
            wait_scatter(b, j - _NBUF)
        relay(b)
        fire_scatter(b, j)
        if not last:
            fire_gather((b + _D) % _NBUF, j + _D)

    for b in range(_NBUF):
        chunk(b, b, first=True)

    @pl.loop(_NBUF, nj - _NBUF, step=_NBUF)
    def _r(j0):
        for b in range(_NBUF):
            chunk(j0 + b, b)

    for b in range(_NBUF):
        j = nj - _NBUF + b
        chunk(j, b, last=(j + _D >= nj))
    for b in range(_NBUF):
        wait_scatter(b, nj - _NBUF + b)


@jax.jit
def _run(idx, table32):
    b, h = idx.shape
    nrows_w = b // _NW
    f = pl.kernel(
        _gather_body,
        out_type=jax.ShapeDtypeStruct((b, h, 2 * _DIM), jnp.bfloat16),
        mesh=plsc.VectorSubcoreMesh(core_axis_name="c", subcore_axis_name="s"),
        scratch_types=[
            pltpu.VMEM((nrows_w, h), jnp.int32),
            pltpu.VMEM((_NBUF, h, _DIM), jnp.int32),
            pltpu.VMEM((_NBUF, h, 2 * _DIM), jnp.bfloat16),
        ] + [pltpu.SemaphoreType.DMA] * (2 * _NBUF),
        compiler_params=pltpu.CompilerParams(
            use_tc_tiling_on_sc=False, needs_layout_passes=False),
    )
    return f(idx, table32)


def kernel(input, weight):
    b, h = input.shape
    assert b % _NW == 0 and (b // _NW) % _NBUF == 0
    nrows, dim = weight.shape
    w32 = jax.lax.bitcast_convert_type(
        weight.reshape(nrows, dim // 2, 2), jnp.int32)
    return _run(input.astype(jnp.int32), w32)


# speedup vs baseline: 1.5880x; 1.0467x over previous
import jax
import jax.numpy as jnp
from jax import lax
from jax.experimental import pallas as pl
from jax.experimental.pallas import tpu as pltpu
from jax.experimental.pallas import tpu_sc as plsc

_DIM = 16
_NC = 2
_NS = 16
_NW = _NC * _NS
_NBUF = 8
_D = 4


def _gather_body(idx_hbm, table_hbm, out_hbm, idx_v, bufg, bufs, *sems):
    gsem = sems[:_NBUF]
    ssem = sems[_NBUF:]
    nrows_w = idx_v.shape[0]
    hist = idx_v.shape[1]
    nj = nrows_w
    wid = lax.axis_index("s") * _NC + lax.axis_index("c")
    wr0 = wid * nrows_w


    pltpu.sync_copy(idx_hbm.at[pl.ds(wr0, nrows_w), :], idx_v)

    def fire_gather(b, j):
        pltpu.async_copy(table_hbm.at[idx_v.at[j]], bufg.at[b], gsem[b])

    def wait_gather(b, j):
        pltpu.make_async_copy(table_hbm.at[idx_v.at[j]], bufg.at[b], gsem[b]).wait()

    def fire_scatter(b, j):
        pltpu.async_copy(bufs.at[b], out_hbm.at[wr0 + j], ssem[b])

    def wait_scatter(b, j):
        pltpu.make_async_copy(bufs.at[b], out_hbm.at[wr0 + j], ssem[b]).wait()

    def relay(b):
        for i in range(hist):
            bufs[b, i, :] = plsc.bitcast(bufg[b, i, :], jnp.bfloat16)

    for j in range(_D):
        fire_gather(j % _NBUF, j)

    def chunk(j, b, first=False, last=False):
        wait_gather(b, j)
        if not first:
            wait_scatter(b, j - _NBUF)
        relay(b)
        fire_scatter(b, j)
        if not last:
            fire_gather((b + _D) % _NBUF, j + _D)

    for b in range(_NBUF):
        chunk(b, b, first=True)

    @pl.loop(_NBUF, nj - _NBUF, step=_NBUF)
    def _r(j0):
        for b in range(_NBUF):
            chunk(j0 + b, b)

    for b in range(_NBUF):
        j = nj - _NBUF + b
        chunk(j, b, last=(j + _D >= nj))
    for b in range(_NBUF):
        wait_scatter(b, nj - _NBUF + b)


@jax.jit
def _run(idx, table32):
    b, h = idx.shape
    nrows_w = b // _NW
    f = pl.kernel(
        _gather_body,
        out_type=jax.ShapeDtypeStruct((b, h, 2 * _DIM), jnp.bfloat16),
        mesh=plsc.VectorSubcoreMesh(core_axis_name="c", subcore_axis_name="s"),
        scratch_types=[
            pltpu.VMEM((nrows_w, h), jnp.int32),
            pltpu.VMEM((_NBUF, h, _DIM), jnp.int32),
            pltpu.VMEM((_NBUF, h, 2 * _DIM), jnp.bfloat16),
        ] + [pltpu.SemaphoreType.DMA] * (2 * _NBUF),
        compiler_params=pltpu.CompilerParams(
            use_tc_tiling_on_sc=False, needs_layout_passes=False),
    )
    return f(idx, table32)


def kernel(input, weight):
    b, h = input.shape
    assert b % _NW == 0 and (b // _NW) % _NBUF == 0
    nrows, dim = weight.shape
    w32 = jax.lax.bitcast_convert_type(
        weight.reshape(nrows, dim // 2, 2), jnp.int32)
    out = _run(input.astype(jnp.int32), w32)
    from jax.experimental import layout as _jl
    lay = _jl.Layout(major_to_minor=(0, 1, 2), tiling=((16,), (1024,)))
    return _jl.with_layout_constraint(out, lay)
